# baseline (speedup 1.0000x reference)
import functools

import jax
import jax.numpy as jnp
from jax import lax
from jax.experimental import pallas as pl
from jax.experimental.pallas import tpu as pltpu
from jax.experimental.pallas import tpu_sc as plsc

NP = 8
NPTS = 500
C = 64
N2 = 2048

L = 16
NCHUNK = C // L
C_PAD = 128
NW = 32
PTS_W = 16
ROWS_W = PTS_W * NP
HALF = ROWS_W // 2
LAST_PT = NPTS - PTS_W


def _make_sc_kernel():
    mesh = plsc.VectorSubcoreMesh(core_axis_name="c", subcore_axis_name="s")

    @functools.partial(
        pl.kernel,
        mesh=mesh,
        out_type=jax.ShapeDtypeStruct((NW, PTS_W * C), jnp.float32),
        scratch_types=[
            pltpu.VMEM((2, HALF), jnp.int32),
            pltpu.VMEM((ROWS_W,), jnp.float32),
            pltpu.VMEM((ROWS_W, C_PAD), jnp.float32),
            pltpu.VMEM((PTS_W * C,), jnp.float32),
            pltpu.SemaphoreType.DMA,
            pltpu.SemaphoreType.DMA,
        ],
    )
    def sc_kernel(table_hbm, idx_hbm, w_hbm, g_hbm,
                  idx_v, w_v, rows_v, g_v, sem0, sem1):
        wid = lax.axis_index("s") * 2 + lax.axis_index("c")
        row_base = jnp.minimum(wid * ROWS_W, NPTS * NP - ROWS_W)

        pltpu.sync_copy(idx_hbm.at[pl.ds(row_base, HALF)], idx_v.at[0])
        cp0 = pltpu.async_copy(
            table_hbm.at[idx_v.at[0]],
            rows_v.at[pl.ds(0, HALF)], sem0)
        pltpu.sync_copy(idx_hbm.at[pl.ds(row_base + HALF, HALF)], idx_v.at[1])
        cp1 = pltpu.async_copy(
            table_hbm.at[idx_v.at[1]],
            rows_v.at[pl.ds(HALF, HALF)], sem1)
        pltpu.sync_copy(w_hbm.at[pl.ds(row_base, ROWS_W)], w_v)

        def pair_body(q, carry):
            wv = w_v[pl.ds(q * 2 * NP, L)]
            for half in range(2):
                p = q * 2 + half
                acc = [jnp.zeros((L,), jnp.float32) for _ in range(NCHUNK)]
                for j in range(NP):
                    k = p * NP + j
                    ws = _lane_splat(wv, half * NP + j)
                    for ch in range(NCHUNK):
                        acc[ch] = acc[ch] + ws * rows_v[k, pl.ds(ch * L, L)]
                for ch in range(NCHUNK):
                    g_v[pl.ds(p * C + ch * L, L)] = acc[ch]
            return carry

        cp0.wait()
        lax.fori_loop(0, PTS_W // 4, pair_body, 0)
        cp1.wait()
        lax.fori_loop(PTS_W // 4, PTS_W // 2, pair_body, 0)

        pltpu.sync_copy(g_v, g_hbm.at[wid])

    return sc_kernel


_SPLAT_DNUMS = lax.GatherDimensionNumbers(
    offset_dims=(), collapsed_slice_dims=(0,), start_index_map=(0,))


def _lane_splat(vec, lane):
    idx = jnp.full((L, 1), lane, jnp.int32)
    return lax.gather(vec, idx, _SPLAT_DNUMS, slice_sizes=(1,),
                      mode=lax.GatherScatterMode.PROMISE_IN_BOUNDS)


_sc_kernel = _make_sc_kernel()


@jax.jit
def kernel(feat1, feat2, inds, weight):
    table = jnp.zeros((N2, C_PAD), jnp.float32)
    table = table.at[:, :C].set(feat2[0].T)
    idx = inds.reshape(-1).astype(jnp.int32)
    w = weight.reshape(-1)

    g = _sc_kernel(table, idx, w)
    g = g.reshape(NW, PTS_W, C)
    gp = jnp.concatenate(
        [g[:NW - 1].reshape((NW - 1) * PTS_W, C),
         g[NW - 1, (NW - 1) * PTS_W - LAST_PT:]], axis=0)

    wsum = jnp.sum(weight.reshape(NPTS, NP), axis=1)
    return (feat1 * wsum[None, None, :] - gp.T[None]) * (1.0 / NP)


# speedup vs baseline: 1.0229x; 1.0229x over previous
import functools

import jax
import jax.numpy as jnp
from jax import lax
from jax.experimental import pallas as pl
from jax.experimental.pallas import tpu as pltpu
from jax.experimental.pallas import tpu_sc as plsc

NP = 8
NPTS = 500
C = 64
N2 = 2048

L = 16
NCHUNK = C // L
C_PAD = 128
NW = 16
PTS_W = 32
ROWS_W = PTS_W * NP
HALF = ROWS_W // 2
LAST_PT = NPTS - PTS_W


def _make_sc_kernel():
    mesh = plsc.VectorSubcoreMesh(core_axis_name="c", subcore_axis_name="s",
                                  num_cores=1)

    @functools.partial(
        pl.kernel,
        mesh=mesh,
        out_type=jax.ShapeDtypeStruct((NW, PTS_W * C), jnp.float32),
        scratch_types=[
            pltpu.VMEM((2, HALF), jnp.int32),
            pltpu.VMEM((ROWS_W,), jnp.float32),
            pltpu.VMEM((ROWS_W, C_PAD), jnp.float32),
            pltpu.VMEM((PTS_W * C,), jnp.float32),
            pltpu.SemaphoreType.DMA,
            pltpu.SemaphoreType.DMA,
        ],
    )
    def sc_kernel(table_hbm, idx_hbm, w_hbm, g_hbm,
                  idx_v, w_v, rows_v, g_v, sem0, sem1):
        wid = lax.axis_index("s")
        row_base = jnp.minimum(wid * ROWS_W, NPTS * NP - ROWS_W)

        pltpu.sync_copy(idx_hbm.at[pl.ds(row_base, HALF)], idx_v.at[0])
        cp0 = pltpu.async_copy(
            table_hbm.at[idx_v.at[0]],
            rows_v.at[pl.ds(0, HALF)], sem0)
        pltpu.sync_copy(idx_hbm.at[pl.ds(row_base + HALF, HALF)], idx_v.at[1])
        cp1 = pltpu.async_copy(
            table_hbm.at[idx_v.at[1]],
            rows_v.at[pl.ds(HALF, HALF)], sem1)
        pltpu.sync_copy(w_hbm.at[pl.ds(row_base, ROWS_W)], w_v)

        def pair_body(q):
            wv = w_v[pl.ds(q * 2 * NP, L)]
            for half in range(2):
                p = q * 2 + half
                acc = [jnp.zeros((L,), jnp.float32) for _ in range(NCHUNK)]
                for j in range(NP):
                    k = p * NP + j
                    ws = _lane_splat(wv, half * NP + j)
                    for ch in range(NCHUNK):
                        acc[ch] = acc[ch] + ws * rows_v[k, pl.ds(ch * L, L)]
                for ch in range(NCHUNK):
                    g_v[pl.ds(p * C + ch * L, L)] = acc[ch]

        cp0.wait()
        plsc.parallel_loop(0, PTS_W // 4, 1, unroll=2)(pair_body)
        cp1.wait()
        plsc.parallel_loop(PTS_W // 4, PTS_W // 2, 1, unroll=2)(pair_body)

        pltpu.sync_copy(g_v, g_hbm.at[wid])

    return sc_kernel


_SPLAT_DNUMS = lax.GatherDimensionNumbers(
    offset_dims=(), collapsed_slice_dims=(0,), start_index_map=(0,))


def _lane_splat(vec, lane):
    idx = jnp.full((L, 1), lane, jnp.int32)
    return lax.gather(vec, idx, _SPLAT_DNUMS, slice_sizes=(1,),
                      mode=lax.GatherScatterMode.PROMISE_IN_BOUNDS)


_sc_kernel = _make_sc_kernel()


@jax.jit
def kernel(feat1, feat2, inds, weight):
    table = jnp.zeros((N2, C_PAD), jnp.float32)
    table = table.at[:, :C].set(feat2[0].T)
    idx = inds.reshape(-1).astype(jnp.int32)
    w = weight.reshape(-1)

    g = _sc_kernel(table, idx, w)
    g = g.reshape(NW, PTS_W, C)
    gp = jnp.concatenate(
        [g[:NW - 1].reshape((NW - 1) * PTS_W, C),
         g[NW - 1, (NW - 1) * PTS_W - LAST_PT:]], axis=0)

    wsum = jnp.sum(weight.reshape(NPTS, NP), axis=1)
    return (feat1 * wsum[None, None, :] - gp.T[None]) * (1.0 / NP)


# speedup vs baseline: 1.1381x; 1.1126x over previous
import functools

import jax
import jax.numpy as jnp
from jax import lax
from jax.experimental import pallas as pl
from jax.experimental.pallas import tpu as pltpu
from jax.experimental.pallas import tpu_sc as plsc

NP = 8
NPTS = 500
C = 64
N2 = 2048

L = 16
NCHUNK = C // L
C_PAD = 128
NW = 16
PTS_W = 32
ROWS_W = PTS_W * NP
HALF = ROWS_W // 2
LAST_PT = NPTS - PTS_W


def _make_sc_kernel():
    mesh = plsc.VectorSubcoreMesh(core_axis_name="c", subcore_axis_name="s",
                                  num_cores=1)

    @functools.partial(
        pl.kernel,
        mesh=mesh,
        out_type=jax.ShapeDtypeStruct((NW, PTS_W * C), jnp.float32),
        scratch_types=[
            pltpu.VMEM((2, HALF), jnp.int32),
            pltpu.VMEM((ROWS_W,), jnp.float32),
            pltpu.VMEM((ROWS_W, C), jnp.float32),
            pltpu.VMEM((PTS_W * C,), jnp.float32),
            pltpu.SemaphoreType.DMA,
            pltpu.SemaphoreType.DMA,
        ],
        compiler_params=pltpu.CompilerParams(use_tc_tiling_on_sc=False),
    )
    def sc_kernel(table_hbm, idx_hbm, w_hbm, g_hbm,
                  idx_v, w_v, rows_v, g_v, sem0, sem1):
        wid = lax.axis_index("s")
        row_base = jnp.minimum(wid * ROWS_W, NPTS * NP - ROWS_W)

        pltpu.sync_copy(idx_hbm.at[pl.ds(row_base, HALF)], idx_v.at[0])
        cp0 = pltpu.async_copy(
            table_hbm.at[idx_v.at[0]],
            rows_v.at[pl.ds(0, HALF)], sem0)
        pltpu.sync_copy(idx_hbm.at[pl.ds(row_base + HALF, HALF)], idx_v.at[1])
        cp1 = pltpu.async_copy(
            table_hbm.at[idx_v.at[1]],
            rows_v.at[pl.ds(HALF, HALF)], sem1)
        pltpu.sync_copy(w_hbm.at[pl.ds(row_base, ROWS_W)], w_v)

        def pair_body(q):
            wv = w_v[pl.ds(q * 2 * NP, L)]
            for half in range(2):
                p = q * 2 + half
                acc = [jnp.zeros((L,), jnp.float32) for _ in range(NCHUNK)]
                for j in range(NP):
                    k = p * NP + j
                    ws = _lane_splat(wv, half * NP + j)
                    for ch in range(NCHUNK):
                        acc[ch] = acc[ch] + ws * rows_v[k, pl.ds(ch * L, L)]
                for ch in range(NCHUNK):
                    g_v[pl.ds(p * C + ch * L, L)] = acc[ch]

        cp0.wait()
        plsc.parallel_loop(0, PTS_W // 4, 1, unroll=2)(pair_body)
        cp1.wait()
        plsc.parallel_loop(PTS_W // 4, PTS_W // 2, 1, unroll=2)(pair_body)

        pltpu.sync_copy(g_v, g_hbm.at[wid])

    return sc_kernel


_SPLAT_DNUMS = lax.GatherDimensionNumbers(
    offset_dims=(), collapsed_slice_dims=(0,), start_index_map=(0,))


def _lane_splat(vec, lane):
    idx = jnp.full((L, 1), lane, jnp.int32)
    return lax.gather(vec, idx, _SPLAT_DNUMS, slice_sizes=(1,),
                      mode=lax.GatherScatterMode.PROMISE_IN_BOUNDS)


_sc_kernel = _make_sc_kernel()


@jax.jit
def kernel(feat1, feat2, inds, weight):
    table = feat2[0].T
    idx = inds.reshape(-1).astype(jnp.int32)
    w = weight.reshape(-1)

    g = _sc_kernel(table, idx, w)
    g = g.reshape(NW, PTS_W, C)
    gp = jnp.concatenate(
        [g[:NW - 1].reshape((NW - 1) * PTS_W, C),
         g[NW - 1, (NW - 1) * PTS_W - LAST_PT:]], axis=0)

    wsum = jnp.sum(weight.reshape(NPTS, NP), axis=1)
    return (feat1 * wsum[None, None, :] - gp.T[None]) * (1.0 / NP)


# speedup vs baseline: 1.1887x; 1.0445x over previous
import functools

import jax
import jax.numpy as jnp
from jax import lax
from jax.experimental import pallas as pl
from jax.experimental.pallas import tpu as pltpu
from jax.experimental.pallas import tpu_sc as plsc

NP = 8
NPTS = 500
C = 64
N2 = 2048

L = 16
NCHUNK = C // L
C_PAD = 128
NW = 16
PTS_W = 32
ROWS_W = PTS_W * NP
HALF = ROWS_W // 2
LAST_PT = NPTS - PTS_W


def _make_sc_kernel():
    mesh = plsc.VectorSubcoreMesh(core_axis_name="c", subcore_axis_name="s",
                                  num_cores=1)

    @functools.partial(
        pl.kernel,
        mesh=mesh,
        out_type=jax.ShapeDtypeStruct((NPTS, C), jnp.float32),
        scratch_types=[
            pltpu.VMEM((2, HALF), jnp.int32),
            pltpu.VMEM((ROWS_W,), jnp.float32),
            pltpu.VMEM((ROWS_W, C), jnp.float32),
            pltpu.VMEM((PTS_W, C), jnp.float32),
            pltpu.SemaphoreType.DMA,
            pltpu.SemaphoreType.DMA,
        ],
        compiler_params=pltpu.CompilerParams(use_tc_tiling_on_sc=False),
    )
    def sc_kernel(table_hbm, idx_hbm, w_hbm, g_hbm,
                  idx_v, w_v, rows_v, g_v, sem0, sem1):
        wid = lax.axis_index("s")
        row_base = jnp.minimum(wid * ROWS_W, NPTS * NP - ROWS_W)

        pltpu.sync_copy(idx_hbm.at[pl.ds(row_base, HALF)], idx_v.at[0])
        cp0 = pltpu.async_copy(
            table_hbm.at[idx_v.at[0]],
            rows_v.at[pl.ds(0, HALF)], sem0)
        pltpu.sync_copy(idx_hbm.at[pl.ds(row_base + HALF, HALF)], idx_v.at[1])
        cp1 = pltpu.async_copy(
            table_hbm.at[idx_v.at[1]],
            rows_v.at[pl.ds(HALF, HALF)], sem1)
        pltpu.sync_copy(w_hbm.at[pl.ds(row_base, ROWS_W)], w_v)

        def pair_body(q):
            wv = w_v[pl.ds(q * 2 * NP, L)]
            for half in range(2):
                p = q * 2 + half
                acc = [jnp.zeros((L,), jnp.float32) for _ in range(NCHUNK)]
                for j in range(NP):
                    k = p * NP + j
                    ws = _lane_splat(wv, half * NP + j)
                    for ch in range(NCHUNK):
                        acc[ch] = acc[ch] + ws * rows_v[k, pl.ds(ch * L, L)]
                for ch in range(NCHUNK):
                    g_v[p, pl.ds(ch * L, L)] = acc[ch]

        cp0.wait()
        plsc.parallel_loop(0, PTS_W // 4, 1, unroll=2)(pair_body)
        cp1.wait()
        plsc.parallel_loop(PTS_W // 4, PTS_W // 2, 1, unroll=2)(pair_body)

        pltpu.sync_copy(g_v, g_hbm.at[pl.ds(row_base // NP, PTS_W)])

    return sc_kernel


_SPLAT_DNUMS = lax.GatherDimensionNumbers(
    offset_dims=(), collapsed_slice_dims=(0,), start_index_map=(0,))


def _lane_splat(vec, lane):
    idx = jnp.full((L, 1), lane, jnp.int32)
    return lax.gather(vec, idx, _SPLAT_DNUMS, slice_sizes=(1,),
                      mode=lax.GatherScatterMode.PROMISE_IN_BOUNDS)


_sc_kernel = _make_sc_kernel()


@jax.jit
def kernel(feat1, feat2, inds, weight):
    table = feat2[0].T
    idx = inds.reshape(-1).astype(jnp.int32)
    w = weight.reshape(-1)

    g = _sc_kernel(table, idx, w)

    wsum = jnp.sum(weight.reshape(NPTS, NP), axis=1)
    return (feat1 * wsum[None, None, :] - g.T[None]) * (1.0 / NP)
